# baseline (speedup 1.0000x reference)
import functools
import math

import jax
import jax.numpy as jnp
from jax import lax
from jax.experimental import pallas as pl
from jax.experimental.pallas import tpu as pltpu
from jax.experimental.pallas import tpu_sc as plsc

D_MODEL = 64
SCALE = math.sqrt(D_MODEL)
NUM_WORKERS = 32
CHUNK = 512


def _sc_embed(idx_flat, table, b_total):
    b_per_w = b_total // NUM_WORKERS
    n_chunks = b_per_w // CHUNK
    mesh = plsc.VectorSubcoreMesh(core_axis_name="c", subcore_axis_name="s")

    @functools.partial(
        pl.kernel,
        out_type=jax.ShapeDtypeStruct((b_total, D_MODEL), jnp.float32),
        mesh=mesh,
        scratch_types=[
            pltpu.VMEM((CHUNK,), jnp.int32),
            pltpu.VMEM((CHUNK, D_MODEL), jnp.float32),
            pltpu.SemaphoreType.DMA,
        ],
        compiler_params=pltpu.CompilerParams(use_tc_tiling_on_sc=False),
    )
    def k(idx_hbm, table_hbm, out_hbm, idx_v, rows_v, sem):
        wid = lax.axis_index("s") * 2 + lax.axis_index("c")
        base = wid * b_per_w

        def chunk_body(c, carry):
            off = base + c * CHUNK
            pltpu.sync_copy(idx_hbm.at[pl.ds(off, CHUNK)], idx_v)
            pltpu.async_copy(table_hbm.at[idx_v], rows_v, sem).wait()

            def scale_row(i, carry2):
                for j in range(D_MODEL // 16):
                    s = pl.ds(j * 16, 16)
                    rows_v[i, s] = rows_v[i, s] * SCALE
                return carry2

            lax.fori_loop(0, CHUNK, scale_row, 0, unroll=4)
            pltpu.sync_copy(rows_v, out_hbm.at[pl.ds(off, CHUNK)])
            return carry

        lax.fori_loop(0, n_chunks, chunk_body, 0)

    return k(idx_flat, table)


def kernel(x, table):
    b_total = x.shape[0] * x.shape[1]
    idx_flat = x.reshape(b_total).astype(jnp.int32)
    out = _sc_embed(idx_flat, table, b_total)
    return out.reshape(x.shape[0], x.shape[1], D_MODEL)


# speedup vs baseline: 1.0890x; 1.0890x over previous
import functools
import math

import jax
import jax.numpy as jnp
from jax import lax
from jax.experimental import pallas as pl
from jax.experimental.pallas import tpu as pltpu
from jax.experimental.pallas import tpu_sc as plsc

D_MODEL = 64
SCALE = math.sqrt(D_MODEL)
NUM_WORKERS = 32
CHUNK = 512


def _sc_embed(idx_flat, table, b_total):
    b_per_w = b_total // NUM_WORKERS
    n_chunks = b_per_w // CHUNK
    assert n_chunks % 2 == 0
    mesh = plsc.VectorSubcoreMesh(core_axis_name="c", subcore_axis_name="s")

    @functools.partial(
        pl.kernel,
        out_type=jax.ShapeDtypeStruct((b_total, D_MODEL), jnp.float32),
        mesh=mesh,
        scratch_types=[
            pltpu.VMEM((b_per_w,), jnp.int32),
            pltpu.VMEM((CHUNK, D_MODEL), jnp.float32),
            pltpu.VMEM((CHUNK, D_MODEL), jnp.float32),
            pltpu.SemaphoreType.DMA,
            pltpu.SemaphoreType.DMA,
            pltpu.SemaphoreType.DMA,
            pltpu.SemaphoreType.DMA,
        ],
        compiler_params=pltpu.CompilerParams(use_tc_tiling_on_sc=False),
    )
    def k(idx_hbm, table_hbm, out_hbm, idx_slab, rows0, rows1,
          gsem0, gsem1, ssem0, ssem1):
        rows = (rows0, rows1)
        gsem = (gsem0, gsem1)
        ssem = (ssem0, ssem1)
        wid = lax.axis_index("s") * 2 + lax.axis_index("c")
        base = wid * b_per_w

        pltpu.sync_copy(idx_hbm.at[pl.ds(base, b_per_w)], idx_slab)

        def start_gather(c, b):
            pltpu.async_copy(
                table_hbm.at[idx_slab.at[pl.ds(c * CHUNK, CHUNK)]],
                rows[b], gsem[b])

        def scale_buf(b):
            def scale_row(i, carry2):
                for j in range(D_MODEL // 16):
                    s = pl.ds(j * 16, 16)
                    rows[b][i, s] = rows[b][i, s] * SCALE
                return carry2
            lax.fori_loop(0, CHUNK, scale_row, 0, unroll=4)

        def start_store(c, b):
            pltpu.async_copy(rows[b], out_hbm.at[pl.ds(base + c * CHUNK, CHUNK)],
                             ssem[b])

        def wait_gather(b):
            pltpu.make_async_copy(
                table_hbm.at[idx_slab.at[pl.ds(0, CHUNK)]], rows[b],
                gsem[b]).wait()

        def wait_store(c, b):
            pltpu.make_async_copy(rows[b], out_hbm.at[pl.ds(base, CHUNK)],
                                  ssem[b]).wait()

        start_gather(0, 0)

        def outer(g, carry):
            for b in (0, 1):
                c = 2 * g + b
                other = 1 - b
                wait_gather(b)
                if b == 1:
                    wait_store(c - 1, other)
                else:
                    @pl.when(g > 0)
                    def _():
                        wait_store(c - 1, other)
                if b == 0:
                    start_gather(c + 1, other)
                else:
                    @pl.when(2 * g + 2 < n_chunks)
                    def _():
                        start_gather(c + 1, other)
                scale_buf(b)
                start_store(c, b)
            return carry

        lax.fori_loop(0, n_chunks // 2, outer, 0)
        wait_store(n_chunks - 1, 1)

    return k(idx_flat, table)


def kernel(x, table):
    b_total = x.shape[0] * x.shape[1]
    idx_flat = x.reshape(b_total).astype(jnp.int32)
    out = _sc_embed(idx_flat, table, b_total)
    return out.reshape(x.shape[0], x.shape[1], D_MODEL)
